# Initial kernel scaffold; baseline (speedup 1.0000x reference)
import jax
import jax.numpy as jnp
from jax.experimental import pallas as pl


def kernel(x, edge_index, edge_attr, u, batch, W1, b1, W2, b2):
    raise NotImplementedError("write your pallas kernel here")


# speedup vs baseline: 6.3669x; 6.3669x over previous
import functools

import jax
import jax.numpy as jnp
from jax import lax
from jax.experimental import pallas as pl
from jax.experimental.pallas import tpu as pltpu
from jax.experimental.pallas import tpu_sc as plsc

_N = 10000
_E = 320000
_DE = 16
_DF = 128
_DH = 128
_DO = 128
_NG = 64

_NC = 2
_NS = 16
_NW = _NC * _NS
_EPW = _E // _NW
_CH = 2000
_NCHUNK = _EPW // _CH
_IB = 80
_KPC = _CH // _IB
_IROWS = _EPW // _IB
_NPAD = 10240
_CPAD = 640
_SROWS = _NPAD // _NS
_CROWS = _CPAD // _NS

_TR = 1000


def _sc_body(ea_hbm, src_hbm, sums_hbm, cnts_hbm,
             ea_buf, idx_buf, ones_buf, sums_sh, cnts_sh, sem):
    cid = lax.axis_index("c")
    sid = lax.axis_index("s")
    w = sid * _NC + cid

    def _zero(i, _):
        ones_buf[i, :] = jnp.zeros((16,), jnp.float32)
        return 0

    lax.fori_loop(0, _SROWS, _zero, 0)
    pltpu.sync_copy(ones_buf.at[pl.ds(0, _SROWS)],
                    sums_sh.at[pl.ds(sid * _SROWS, _SROWS)])
    pltpu.sync_copy(ones_buf.at[pl.ds(0, _SROWS)],
                    cnts_sh.at[pl.ds(sid * _SROWS, _SROWS)])

    def _ones(i, _):
        ones_buf[i, :] = jnp.ones((16,), jnp.float32)
        return 0

    lax.fori_loop(0, _CH, _ones, 0)
    plsc.subcore_barrier()

    pltpu.sync_copy(src_hbm.at[w], idx_buf)

    for k in range(_NCHUNK):
        e0 = w * _EPW + k * _CH
        pltpu.sync_copy(ea_hbm.at[pl.ds(e0, _CH)], ea_buf)

        def _fire(t, _):
            pltpu.async_copy(ea_buf.at[pl.ds(t * _IB, _IB)],
                             sums_sh.at[idx_buf.at[k * _KPC + t]], sem,
                             add=True)
            return 0

        lax.fori_loop(0, _KPC, _fire, 0)

        def _drain(t, _):
            pltpu.make_async_copy(ea_buf.at[pl.ds(t * _IB, _IB)],
                                  sums_sh.at[idx_buf.at[k * _KPC + t]],
                                  sem).wait()
            return 0

        lax.fori_loop(0, _KPC, _drain, 0)

        def _cfire(t, _):
            pltpu.async_copy(ones_buf.at[pl.ds(t * _IB, _IB)],
                             cnts_sh.at[idx_buf.at[k * _KPC + t]], sem,
                             add=True)
            return 0

        lax.fori_loop(0, _KPC, _cfire, 0)

        def _cdrain(t, _):
            pltpu.make_async_copy(ones_buf.at[pl.ds(t * _IB, _IB)],
                                  cnts_sh.at[idx_buf.at[k * _KPC + t]],
                                  sem).wait()
            return 0

        lax.fori_loop(0, _KPC, _cdrain, 0)

    plsc.subcore_barrier()

    pltpu.sync_copy(sums_sh.at[pl.ds(sid * _SROWS, _SROWS)],
                    sums_hbm.at[cid, pl.ds(sid * _SROWS, _SROWS)])
    pltpu.sync_copy(cnts_sh.at[pl.ds(sid * _SROWS, _SROWS)],
                    cnts_hbm.at[cid, pl.ds(sid * _SROWS, _SROWS)])


def _scatter_sc(edge_attr, src2):
    mesh = plsc.VectorSubcoreMesh(core_axis_name="c", subcore_axis_name="s")
    f = pl.kernel(
        _sc_body,
        out_type=[
            jax.ShapeDtypeStruct((_NC, _NPAD, _DE), jnp.float32),
            jax.ShapeDtypeStruct((_NC, _NPAD, 16), jnp.float32),
        ],
        mesh=mesh,
        scratch_types=[
            pltpu.VMEM((_CH, _DE), jnp.float32),
            pltpu.VMEM((_IROWS, _IB), jnp.int32),
            pltpu.VMEM((_CH, 16), jnp.float32),
            pltpu.VMEM_SHARED((_NPAD, _DE), jnp.float32),
            pltpu.VMEM_SHARED((_NPAD, 16), jnp.float32),
            pltpu.SemaphoreType.DMA,
        ],
        compiler_params=pltpu.CompilerParams(needs_layout_passes=False,
                                             use_tc_tiling_on_sc=False),
    )
    return f(edge_attr, src2)


def _tc_body(x_ref, s_ref, c_ref, b_ref, u_ref, w1_ref, b1_ref, w2_ref,
             b2_ref, o_ref):
    s = s_ref[0] + s_ref[1]
    c = c_ref[0, :, 0:1] + c_ref[1, :, 0:1]
    r = 1.0 / jnp.maximum(c, 1.0)
    xw = jnp.dot(x_ref[...], w1_ref[0:_DF, :],
                 preferred_element_type=jnp.float32)
    vew = jnp.dot(s, w1_ref[_DF:_DF + _DE, :],
                  preferred_element_type=jnp.float32) * r
    uwb = jnp.dot(u_ref[...], w1_ref[_DF + _DE:, :],
                  preferred_element_type=jnp.float32) + b1_ref[...]
    oh = (b_ref[...] == lax.broadcasted_iota(jnp.int32, (1, _NG), 1))
    uemb = jnp.dot(oh.astype(jnp.float32), uwb,
                   preferred_element_type=jnp.float32)
    a = xw + vew + uemb
    h = jnp.maximum(a, 0.0) + jnp.log1p(jnp.exp(-jnp.abs(a)))
    o_ref[...] = jnp.dot(h, w2_ref[...],
                         preferred_element_type=jnp.float32) + b2_ref[...]


def _mlp_tc(x, sums3, cnts3, batch2, u, W1, b1r, W2, b2r):
    grid = (_N // _TR,)
    return pl.pallas_call(
        _tc_body,
        grid=grid,
        in_specs=[
            pl.BlockSpec((_TR, _DF), lambda i: (i, 0)),
            pl.BlockSpec((_NC, _TR, _DE), lambda i: (0, i, 0)),
            pl.BlockSpec((_NC, _TR, 16), lambda i: (0, i, 0)),
            pl.BlockSpec((_TR, 1), lambda i: (i, 0)),
            pl.BlockSpec((_NG, 16), lambda i: (0, 0)),
            pl.BlockSpec((_DF + _DE + 16, _DH), lambda i: (0, 0)),
            pl.BlockSpec((1, _DH), lambda i: (0, 0)),
            pl.BlockSpec((_DH, _DO), lambda i: (0, 0)),
            pl.BlockSpec((1, _DO), lambda i: (0, 0)),
        ],
        out_specs=pl.BlockSpec((_TR, _DO), lambda i: (i, 0)),
        out_shape=jax.ShapeDtypeStruct((_N, _DO), jnp.float32),
    )(x, sums3, cnts3, batch2, u, W1, b1r, W2, b2r)


def kernel(x, edge_index, edge_attr, u, batch, W1, b1, W2, b2):
    src2 = edge_index[0].astype(jnp.int32).reshape(_NW, _IROWS, _IB)
    sums_raw, cnts_raw = _scatter_sc(edge_attr, src2)
    batch2 = batch.astype(jnp.int32).reshape(_N, 1)
    return _mlp_tc(x, sums_raw, cnts_raw, batch2, u, W1, b1.reshape(1, _DH),
                   W2, b2.reshape(1, _DO))


# speedup vs baseline: 6.3768x; 1.0016x over previous
import functools

import jax
import jax.numpy as jnp
from jax import lax
from jax.experimental import pallas as pl
from jax.experimental.pallas import tpu as pltpu
from jax.experimental.pallas import tpu_sc as plsc

_N = 10000
_E = 320000
_DE = 16
_DF = 128
_DH = 128
_DO = 128
_NG = 64

_NC = 2
_NS = 16
_NW = _NC * _NS
_EPW = _E // _NW
_CH = 2000
_NCHUNK = _EPW // _CH
_IB = 80
_KPC = _CH // _IB
_IROWS = _EPW // _IB
_NPAD = 10240
_CPAD = 640
_SROWS = _NPAD // _NS
_CROWS = _CPAD // _NS

_TR = 1000


def _sc_body(ea_hbm, src_hbm, sums_hbm, cnts_hbm,
             ea_buf, idx_buf, ones_buf, sums_sh, cnts_sh, sem):
    cid = lax.axis_index("c")
    sid = lax.axis_index("s")
    w = sid * _NC + cid

    def _zero(i, _):
        ones_buf[i, :] = jnp.zeros((16,), jnp.float32)
        return 0

    lax.fori_loop(0, _SROWS, _zero, 0)
    pltpu.sync_copy(ones_buf.at[pl.ds(0, _SROWS)],
                    sums_sh.at[pl.ds(sid * _SROWS, _SROWS)])
    pltpu.sync_copy(ones_buf.at[pl.ds(0, _SROWS)],
                    cnts_sh.at[pl.ds(sid * _SROWS, _SROWS)])

    def _ones(i, _):
        ones_buf[i, :] = jnp.ones((16,), jnp.float32)
        return 0

    lax.fori_loop(0, _CH, _ones, 0)
    plsc.subcore_barrier()

    pltpu.sync_copy(src_hbm.at[0, w], idx_buf)

    for k in range(_NCHUNK):
        e0 = w * _EPW + k * _CH
        pltpu.sync_copy(ea_hbm.at[pl.ds(e0, _CH)], ea_buf)

        def _fire(t, _):
            pltpu.async_copy(ea_buf.at[pl.ds(t * _IB, _IB)],
                             sums_sh.at[idx_buf.at[k * _KPC + t]], sem,
                             add=True)
            return 0

        lax.fori_loop(0, _KPC, _fire, 0)

        def _drain(t, _):
            pltpu.make_async_copy(ea_buf.at[pl.ds(t * _IB, _IB)],
                                  sums_sh.at[idx_buf.at[k * _KPC + t]],
                                  sem).wait()
            return 0

        lax.fori_loop(0, _KPC, _drain, 0)

        def _cfire(t, _):
            pltpu.async_copy(ones_buf.at[pl.ds(t * _IB, _IB)],
                             cnts_sh.at[idx_buf.at[k * _KPC + t]], sem,
                             add=True)
            return 0

        lax.fori_loop(0, _KPC, _cfire, 0)

        def _cdrain(t, _):
            pltpu.make_async_copy(ones_buf.at[pl.ds(t * _IB, _IB)],
                                  cnts_sh.at[idx_buf.at[k * _KPC + t]],
                                  sem).wait()
            return 0

        lax.fori_loop(0, _KPC, _cdrain, 0)

    plsc.subcore_barrier()

    pltpu.sync_copy(sums_sh.at[pl.ds(sid * _SROWS, _SROWS)],
                    sums_hbm.at[cid, pl.ds(sid * _SROWS, _SROWS)])
    pltpu.sync_copy(cnts_sh.at[pl.ds(sid * _SROWS, _SROWS)],
                    cnts_hbm.at[cid, pl.ds(sid * _SROWS, _SROWS)])


def _scatter_sc(edge_attr, src2):
    mesh = plsc.VectorSubcoreMesh(core_axis_name="c", subcore_axis_name="s")
    f = pl.kernel(
        _sc_body,
        out_type=[
            jax.ShapeDtypeStruct((_NC, _NPAD, _DE), jnp.float32),
            jax.ShapeDtypeStruct((_NC, _NPAD, 16), jnp.float32),
        ],
        mesh=mesh,
        scratch_types=[
            pltpu.VMEM((_CH, _DE), jnp.float32),
            pltpu.VMEM((_IROWS, _IB), jnp.int32),
            pltpu.VMEM((_CH, 16), jnp.float32),
            pltpu.VMEM_SHARED((_NPAD, _DE), jnp.float32),
            pltpu.VMEM_SHARED((_NPAD, 16), jnp.float32),
            pltpu.SemaphoreType.DMA,
        ],
        compiler_params=pltpu.CompilerParams(needs_layout_passes=False,
                                             use_tc_tiling_on_sc=False),
    )
    return f(edge_attr, src2)


def _tc_body(x_ref, s_ref, c_ref, b_ref, u_ref, w1_ref, b1_ref, w2_ref,
             b2_ref, o_ref):
    s = s_ref[0] + s_ref[1]
    c = c_ref[0, :, 0:1] + c_ref[1, :, 0:1]
    r = 1.0 / jnp.maximum(c, 1.0)
    xw = jnp.dot(x_ref[...], w1_ref[0:_DF, :],
                 preferred_element_type=jnp.float32)
    vew = jnp.dot(s, w1_ref[_DF:_DF + _DE, :],
                  preferred_element_type=jnp.float32) * r
    uwb = jnp.dot(u_ref[...], w1_ref[_DF + _DE:, :],
                  preferred_element_type=jnp.float32) + b1_ref[...]
    oh = (b_ref[...] == lax.broadcasted_iota(jnp.int32, (1, _NG), 1))
    uemb = jnp.dot(oh.astype(jnp.float32), uwb,
                   preferred_element_type=jnp.float32)
    a = xw + vew + uemb
    h = jnp.maximum(a, 0.0) + jnp.log1p(jnp.exp(-jnp.abs(a)))
    o_ref[...] = jnp.dot(h, w2_ref[...],
                         preferred_element_type=jnp.float32) + b2_ref[...]


def _mlp_tc(x, sums3, cnts3, batch2, u, W1, b1r, W2, b2r):
    grid = (_N // _TR,)
    return pl.pallas_call(
        _tc_body,
        grid=grid,
        in_specs=[
            pl.BlockSpec((_TR, _DF), lambda i: (i, 0)),
            pl.BlockSpec((_NC, _TR, _DE), lambda i: (0, i, 0)),
            pl.BlockSpec((_NC, _TR, 16), lambda i: (0, i, 0)),
            pl.BlockSpec((_TR, 1), lambda i: (i, 0)),
            pl.BlockSpec((_NG, 16), lambda i: (0, 0)),
            pl.BlockSpec((_DF + _DE + 16, _DH), lambda i: (0, 0)),
            pl.BlockSpec((1, _DH), lambda i: (0, 0)),
            pl.BlockSpec((_DH, _DO), lambda i: (0, 0)),
            pl.BlockSpec((1, _DO), lambda i: (0, 0)),
        ],
        out_specs=pl.BlockSpec((_TR, _DO), lambda i: (i, 0)),
        out_shape=jax.ShapeDtypeStruct((_N, _DO), jnp.float32),
    )(x, sums3, cnts3, batch2, u, W1, b1r, W2, b2r)


def kernel(x, edge_index, edge_attr, u, batch, W1, b1, W2, b2):
    src4 = edge_index.reshape(2, _NW, _IROWS, _IB)
    sums_raw, cnts_raw = _scatter_sc(edge_attr, src4)
    batch2 = batch.astype(jnp.int32).reshape(_N, 1)
    return _mlp_tc(x, sums_raw, cnts_raw, batch2, u, W1, b1.reshape(1, _DH),
                   W2, b2.reshape(1, _DO))
